# baseline (speedup 1.0000x reference)
import functools

import jax
import jax.numpy as jnp
from jax.experimental import pallas as pl
from jax.experimental.pallas import tpu as pltpu

NB = 255


def _body(x_ref, c_ref, o_ref):
    x = x_ref[...]
    c = c_ref[...]
    m = jnp.max(x, axis=1, keepdims=True)
    e = jnp.exp(x - m)
    s = jnp.sum(e, axis=1, keepdims=True)
    w = jnp.sum(e * c, axis=1, keepdims=True)
    v = w / s
    o_ref[...] = jnp.sign(v) * (jnp.exp(jnp.abs(v)) - 1.0)


@functools.partial(jax.jit, static_argnames=())
def kernel(logits, centers):
    n, nb = logits.shape
    br = 2048
    grid = (n // br,)
    c2 = centers.reshape(1, nb)
    out = pl.pallas_call(
        _body,
        grid=grid,
        in_specs=[
            pl.BlockSpec((br, nb), lambda i: (i, 0)),
            pl.BlockSpec((1, nb), lambda i: (0, 0)),
        ],
        out_specs=pl.BlockSpec((br, 1), lambda i: (i, 0)),
        out_shape=jax.ShapeDtypeStruct((n, 1), logits.dtype),
        compiler_params=pltpu.CompilerParams(
            dimension_semantics=("arbitrary",),
        ),
    )(logits, c2)
    return out.reshape(n)


# speedup vs baseline: 1.3441x; 1.3441x over previous
import functools

import jax
import jax.numpy as jnp
from jax.experimental import pallas as pl
from jax.experimental.pallas import tpu as pltpu

NB = 255


def _body(x_ref, c_ref, o_ref):
    x = x_ref[...]
    c = c_ref[...]
    m = jnp.max(x, axis=1, keepdims=True)
    e = jnp.exp(x - m)
    s = jnp.sum(e, axis=1, keepdims=True)
    w = jnp.sum(e * c, axis=1, keepdims=True)
    v = w / s
    o_ref[...] = jnp.sign(v) * (jnp.exp(jnp.abs(v)) - 1.0)


@functools.partial(jax.jit, static_argnames=())
def kernel(logits, centers):
    n, nb = logits.shape
    br = 8192
    grid = (n // br,)
    c2 = centers.reshape(1, nb)
    out = pl.pallas_call(
        _body,
        grid=grid,
        in_specs=[
            pl.BlockSpec((br, nb), lambda i: (i, 0)),
            pl.BlockSpec((1, nb), lambda i: (0, 0)),
        ],
        out_specs=pl.BlockSpec((br, 1), lambda i: (i, 0)),
        out_shape=jax.ShapeDtypeStruct((n, 1), logits.dtype),
        compiler_params=pltpu.CompilerParams(
            dimension_semantics=("parallel",),
        ),
    )(logits, c2)
    return out.reshape(n)
